# Initial kernel scaffold; baseline (speedup 1.0000x reference)
import jax
import jax.numpy as jnp
from jax.experimental import pallas as pl


def kernel(x, edge_index, W1, b1, Wc1, bc1, Wc2, bc2, ln_g, ln_b, Wr, br):
    raise NotImplementedError("write your pallas kernel here")


# speedup vs baseline: 7.1946x; 7.1946x over previous
import functools

import jax
import jax.numpy as jnp
from jax import lax
from jax.experimental import pallas as pl
from jax.experimental.pallas import tpu as pltpu
from jax.experimental.pallas import tpu_sc as plsc

N = 10000
E = 160000
D = 256
H = 256
OUT = 3

NC = 2
NS = 16
NTILES = NC * NS
CHUNK = 128
NCH = 40
EPAD = NTILES * NCH * CHUNK
NCH2 = EPAD // (NS * CHUNK)
NPAD = 10240
HALF = H // 2
RPT = NPAD // NS

def _sc_degree_body(dst_hbm, out_hbm, idx_v, ones_v, zero_v, acc_sh):
    c = lax.axis_index("c")
    s = lax.axis_index("s")
    w = s * NC + c

    for k in range(CHUNK // 16):
        ones_v[pl.ds(k * 16, 16)] = jnp.ones((16,), jnp.float32)

    def _z(i, _):
        zero_v[pl.ds(i * 16, 16)] = jnp.zeros((16,), jnp.float32)
        return 0
    lax.fori_loop(0, RPT // 16, _z, 0)

    pltpu.sync_copy(zero_v, acc_sh.at[pl.ds(s * RPT, RPT)])
    plsc.subcore_barrier()

    pltpu.sync_copy(dst_hbm.at[w], idx_v)

    def _step(j, _):
        pltpu.sync_copy(ones_v, acc_sh.at[idx_v.at[j]], add=True)
        return 0
    lax.fori_loop(0, NCH, _step, 0)

    plsc.subcore_barrier()
    pltpu.sync_copy(acc_sh.at[pl.ds(s * RPT, RPT)],
                    out_hbm.at[c, pl.ds(s * RPT, RPT)])


def _sc_segsum_body(ulo_hbm, uhi_hbm, src_hbm, dst_hbm, outlo_hbm, outhi_hbm,
                    idx_s, idx_d, rows, acc_sh, gsem):
    c = lax.axis_index("c")
    s = lax.axis_index("s")

    def _z(i, _):
        for k in range(HALF // 16):
            rows[0, i, pl.ds(k * 16, 16)] = jnp.zeros((16,), jnp.float32)
        return 0
    lax.fori_loop(0, CHUNK, _z, 0)
    for t in range(RPT // CHUNK):
        pltpu.sync_copy(rows.at[0], acc_sh.at[pl.ds(s * RPT + t * CHUNK, CHUNK)])
    plsc.subcore_barrier()

    def _run(table, out_hbm):
        for p in range(NCH2 // NCH):
            pltpu.sync_copy(src_hbm.at[s, p], idx_s)
            pltpu.sync_copy(dst_hbm.at[s, p], idx_d)

            pltpu.async_copy(table.at[idx_s.at[0]], rows.at[0], gsem)

            def _pair(i, _):
                j0 = i * 2
                for b in range(2):
                    j = j0 + b
                    pltpu.make_async_copy(table.at[idx_s.at[j]], rows.at[b],
                                          gsem).wait()

                    @pl.when(j + 1 < NCH)
                    def _():
                        pltpu.async_copy(table.at[idx_s.at[j + 1]],
                                         rows.at[1 - b], gsem)

                    pltpu.sync_copy(rows.at[b], acc_sh.at[idx_d.at[j]],
                                    add=True)
                return 0
            lax.fori_loop(0, NCH // 2, _pair, 0)

        plsc.subcore_barrier()
        for t in range(RPT // CHUNK):
            r0 = s * RPT + t * CHUNK
            pltpu.sync_copy(acc_sh.at[pl.ds(r0, CHUNK)],
                            out_hbm.at[pl.ds(r0, CHUNK)])

    @pl.when(c == 0)
    def _():
        _run(ulo_hbm, outlo_hbm)

    @pl.when(c == 1)
    def _():
        _run(uhi_hbm, outhi_hbm)


@functools.cache
def _sc_kernels():
    mesh = plsc.VectorSubcoreMesh(core_axis_name="c", subcore_axis_name="s")
    sc_degree = functools.partial(
        pl.kernel,
        out_type=jax.ShapeDtypeStruct((NC, NPAD), jnp.float32),
        mesh=mesh,
        scratch_types=[
            pltpu.VMEM((NCH, CHUNK), jnp.int32),
            pltpu.VMEM((CHUNK,), jnp.float32),
            pltpu.VMEM((RPT,), jnp.float32),
            pltpu.VMEM_SHARED((NPAD,), jnp.float32),
        ],
    )(_sc_degree_body)
    sc_segsum = functools.partial(
        pl.kernel,
        out_type=(jax.ShapeDtypeStruct((NPAD, HALF), jnp.float32),
                  jax.ShapeDtypeStruct((NPAD, HALF), jnp.float32)),
        mesh=mesh,
        scratch_types=[
            pltpu.VMEM((NCH, CHUNK), jnp.int32),
            pltpu.VMEM((NCH, CHUNK), jnp.int32),
            pltpu.VMEM((2, CHUNK, HALF), jnp.float32),
            pltpu.VMEM_SHARED((NPAD, HALF), jnp.float32),
            pltpu.SemaphoreType.DMA,
        ],
    )(_sc_segsum_body)
    return sc_degree, sc_segsum


RB = 1024
GRID = NPAD // RB
_F32 = jnp.float32


def _leaky(x):
    return jnp.where(x >= 0, x, 0.01 * x)


def _dot(a, b):
    return lax.dot_general(a, b, (((1,), (0,)), ((), ())),
                           precision=lax.Precision.HIGHEST,
                           preferred_element_type=_F32)


def _tc_a_body(x_ref, w1_ref, b1_ref, wc1_ref, deg_ref,
               t1_ref, ulo_ref, uhi_ref, dinv_ref):
    i = pl.program_id(0)
    h0 = _leaky(_dot(x_ref[...], w1_ref[...]) + b1_ref[...])
    t1 = _dot(h0, wc1_ref[...])
    deg = deg_ref[0, pl.ds(i * RB, RB)] + deg_ref[1, pl.ds(i * RB, RB)] + 1.0
    dinv = lax.rsqrt(deg).reshape(RB, 1)
    u = t1 * dinv
    t1_ref[...] = t1
    ulo_ref[...] = u[:, :HALF]
    uhi_ref[...] = u[:, HALF:]
    dinv_ref[...] = dinv


def _tc_mid_body(t_ref, slo_ref, shi_ref, dinv_ref, bc_ref, g_ref, b_ref,
                 wc_ref, t2_ref, ulo_ref, uhi_ref):
    dinv = dinv_ref[...]
    S = jnp.concatenate([slo_ref[...], shi_ref[...]], axis=1)
    agg = dinv * S + (dinv * dinv) * t_ref[...] + bc_ref[...]
    m = jnp.mean(agg, axis=1, keepdims=True)
    ctr = agg - m
    v = jnp.mean(ctr * ctr, axis=1, keepdims=True)
    h = ctr * lax.rsqrt(v + 1e-5) * g_ref[...] + b_ref[...]
    h = _leaky(h)
    t2 = _dot(h, wc_ref[...])
    u = t2 * dinv
    t2_ref[...] = t2
    ulo_ref[...] = u[:, :HALF]
    uhi_ref[...] = u[:, HALF:]


def _tc_c_body(t_ref, slo_ref, shi_ref, dinv_ref, bc_ref, g_ref, b_ref,
               wr_ref, br_ref, out_ref):
    dinv = dinv_ref[...]
    S = jnp.concatenate([slo_ref[...], shi_ref[...]], axis=1)
    agg = dinv * S + (dinv * dinv) * t_ref[...] + bc_ref[...]
    m = jnp.mean(agg, axis=1, keepdims=True)
    ctr = agg - m
    v = jnp.mean(ctr * ctr, axis=1, keepdims=True)
    h = ctr * lax.rsqrt(v + 1e-5) * g_ref[...] + b_ref[...]
    h = _leaky(h)
    out_ref[...] = _dot(h, wr_ref[...]) + br_ref[...]


def _row_spec(cols):
    return pl.BlockSpec((RB, cols), lambda i: (i, 0))


def _full_spec(shape):
    nd = len(shape)
    return pl.BlockSpec(shape, lambda i: (0,) * nd)


def kernel(x, edge_index, W1, b1, Wc1, bc1, Wc2, bc2, ln_g, ln_b, Wr, br):
    src = edge_index[0]
    dst = edge_index[1]
    pad = jnp.full((EPAD - E,), N, jnp.int32)
    src_full = jnp.concatenate([src, pad])
    dst_full = jnp.concatenate([dst, pad])
    dst_r = dst_full.reshape(NTILES, NCH, CHUNK)
    src_r2 = src_full.reshape(NS, NCH2 // NCH, NCH, CHUNK)
    dst_r2 = dst_full.reshape(NS, NCH2 // NCH, NCH, CHUNK)
    x_pad = jnp.pad(x, ((0, NPAD - N), (0, 0)))
    b1r = b1.reshape(1, H)
    bc1r = bc1.reshape(1, H)
    bc2r = bc2.reshape(1, H)
    gr = ln_g.reshape(1, H)
    br2 = ln_b.reshape(1, H)
    brr = br.reshape(1, OUT)

    sc_degree, sc_segsum = _sc_kernels()
    deg2 = sc_degree(dst_r)

    t1, u1lo, u1hi, dinv = pl.pallas_call(
        _tc_a_body,
        grid=(GRID,),
        in_specs=[_row_spec(D), _full_spec((D, H)), _full_spec((1, H)),
                  _full_spec((H, H)), _full_spec((NC, NPAD))],
        out_specs=[_row_spec(H), _row_spec(HALF), _row_spec(HALF),
                   _row_spec(1)],
        out_shape=[jax.ShapeDtypeStruct((NPAD, H), _F32),
                   jax.ShapeDtypeStruct((NPAD, HALF), _F32),
                   jax.ShapeDtypeStruct((NPAD, HALF), _F32),
                   jax.ShapeDtypeStruct((NPAD, 1), _F32)],
    )(x_pad, W1, b1r, Wc1, deg2)

    s1lo, s1hi = sc_segsum(u1lo, u1hi, src_r2, dst_r2)

    t2, u2lo, u2hi = pl.pallas_call(
        _tc_mid_body,
        grid=(GRID,),
        in_specs=[_row_spec(H), _row_spec(HALF), _row_spec(HALF),
                  _row_spec(1), _full_spec((1, H)), _full_spec((1, H)),
                  _full_spec((1, H)), _full_spec((H, H))],
        out_specs=[_row_spec(H), _row_spec(HALF), _row_spec(HALF)],
        out_shape=[jax.ShapeDtypeStruct((NPAD, H), _F32),
                   jax.ShapeDtypeStruct((NPAD, HALF), _F32),
                   jax.ShapeDtypeStruct((NPAD, HALF), _F32)],
    )(t1, s1lo, s1hi, dinv, bc1r, gr, br2, Wc2)

    s2lo, s2hi = sc_segsum(u2lo, u2hi, src_r2, dst_r2)

    out_pad = pl.pallas_call(
        _tc_c_body,
        grid=(GRID,),
        in_specs=[_row_spec(H), _row_spec(HALF), _row_spec(HALF),
                  _row_spec(1), _full_spec((1, H)), _full_spec((1, H)),
                  _full_spec((1, H)), _full_spec((H, OUT)),
                  _full_spec((1, OUT))],
        out_specs=_row_spec(OUT),
        out_shape=jax.ShapeDtypeStruct((NPAD, OUT), _F32),
    )(t2, s2lo, s2hi, dinv, bc2r, gr, br2, Wr, brr)

    return out_pad[:N]


# speedup vs baseline: 7.6607x; 1.0648x over previous
import functools

import jax
import jax.numpy as jnp
from jax import lax
from jax.experimental import pallas as pl
from jax.experimental.pallas import tpu as pltpu
from jax.experimental.pallas import tpu_sc as plsc

N = 10000
E = 160000
D = 256
H = 256
OUT = 3

NC = 2
NS = 16
NTILES = NC * NS
CHUNK = 128
NCH = 40
EPAD = NTILES * NCH * CHUNK
NCH2 = EPAD // (NS * CHUNK)
NPAD = 10240
HALF = H // 2
RPT = NPAD // NS

def _sc_degree_body(dst_hbm, out_hbm, idx_v, ones_v, zero_v, acc_sh):
    c = lax.axis_index("c")
    s = lax.axis_index("s")
    w = s * NC + c

    for k in range(CHUNK // 16):
        ones_v[pl.ds(k * 16, 16)] = jnp.ones((16,), jnp.float32)

    def _z(i, _):
        zero_v[pl.ds(i * 16, 16)] = jnp.zeros((16,), jnp.float32)
        return 0
    lax.fori_loop(0, RPT // 16, _z, 0)

    pltpu.sync_copy(zero_v, acc_sh.at[pl.ds(s * RPT, RPT)])
    plsc.subcore_barrier()

    pltpu.sync_copy(dst_hbm.at[w], idx_v)

    def _step(j, _):
        pltpu.sync_copy(ones_v, acc_sh.at[idx_v.at[j]], add=True)
        return 0
    lax.fori_loop(0, NCH, _step, 0)

    plsc.subcore_barrier()
    pltpu.sync_copy(acc_sh.at[pl.ds(s * RPT, RPT)],
                    out_hbm.at[c, pl.ds(s * RPT, RPT)])


def _sc_segsum_body(ulo_hbm, uhi_hbm, src_hbm, dst_hbm, outlo_hbm, outhi_hbm,
                    idx_s, idx_d, rows, acc_sh, gsem, ssem):
    c = lax.axis_index("c")
    s = lax.axis_index("s")

    def _z(i, _):
        for k in range(HALF // 16):
            rows[0, i, pl.ds(k * 16, 16)] = jnp.zeros((16,), jnp.float32)
        return 0
    lax.fori_loop(0, CHUNK, _z, 0)
    for t in range(RPT // CHUNK):
        pltpu.sync_copy(rows.at[0], acc_sh.at[pl.ds(s * RPT + t * CHUNK, CHUNK)])
    plsc.subcore_barrier()

    def _run(table, out_hbm):
        for p in range(NCH2 // NCH):
            pltpu.sync_copy(src_hbm.at[s, p], idx_s)
            pltpu.sync_copy(dst_hbm.at[s, p], idx_d)

            pltpu.async_copy(table.at[idx_s.at[0]], rows.at[0], gsem)

            def _pair(i, _):
                j0 = i * 2
                for b in range(2):
                    j = j0 + b

                    @pl.when(j + 1 < NCH)
                    def _():
                        @pl.when(j >= 1)
                        def _():
                            pltpu.make_async_copy(
                                rows.at[1 - b], acc_sh.at[idx_d.at[j - 1]],
                                ssem).wait()
                        pltpu.async_copy(table.at[idx_s.at[j + 1]],
                                         rows.at[1 - b], gsem)

                    pltpu.make_async_copy(table.at[idx_s.at[j]], rows.at[b],
                                          gsem).wait()
                    pltpu.async_copy(rows.at[b], acc_sh.at[idx_d.at[j]],
                                     ssem, add=True)
                return 0
            lax.fori_loop(0, NCH // 2, _pair, 0)

            for b in range(2):
                pltpu.make_async_copy(rows.at[b], acc_sh.at[idx_d.at[0]],
                                      ssem).wait()

        plsc.subcore_barrier()
        for t in range(RPT // CHUNK):
            r0 = s * RPT + t * CHUNK
            pltpu.sync_copy(acc_sh.at[pl.ds(r0, CHUNK)],
                            out_hbm.at[pl.ds(r0, CHUNK)])

    @pl.when(c == 0)
    def _():
        _run(ulo_hbm, outlo_hbm)

    @pl.when(c == 1)
    def _():
        _run(uhi_hbm, outhi_hbm)


@functools.cache
def _sc_kernels():
    mesh = plsc.VectorSubcoreMesh(core_axis_name="c", subcore_axis_name="s")
    sc_degree = functools.partial(
        pl.kernel,
        out_type=jax.ShapeDtypeStruct((NC, NPAD), jnp.float32),
        mesh=mesh,
        scratch_types=[
            pltpu.VMEM((NCH, CHUNK), jnp.int32),
            pltpu.VMEM((CHUNK,), jnp.float32),
            pltpu.VMEM((RPT,), jnp.float32),
            pltpu.VMEM_SHARED((NPAD,), jnp.float32),
        ],
    )(_sc_degree_body)
    sc_segsum = functools.partial(
        pl.kernel,
        out_type=(jax.ShapeDtypeStruct((NPAD, HALF), jnp.float32),
                  jax.ShapeDtypeStruct((NPAD, HALF), jnp.float32)),
        mesh=mesh,
        scratch_types=[
            pltpu.VMEM((NCH, CHUNK), jnp.int32),
            pltpu.VMEM((NCH, CHUNK), jnp.int32),
            pltpu.VMEM((2, CHUNK, HALF), jnp.float32),
            pltpu.VMEM_SHARED((NPAD, HALF), jnp.float32),
            pltpu.SemaphoreType.DMA,
            pltpu.SemaphoreType.DMA,
        ],
    )(_sc_segsum_body)
    return sc_degree, sc_segsum


RB = 1024
GRID = NPAD // RB
_F32 = jnp.float32


def _leaky(x):
    return jnp.where(x >= 0, x, 0.01 * x)


def _dot(a, b):
    return lax.dot_general(a, b, (((1,), (0,)), ((), ())),
                           precision=lax.Precision.HIGHEST,
                           preferred_element_type=_F32)


def _tc_a_body(x_ref, w1_ref, b1_ref, wc1_ref, deg_ref,
               t1_ref, ulo_ref, uhi_ref, dinv_ref):
    i = pl.program_id(0)
    h0 = _leaky(_dot(x_ref[...], w1_ref[...]) + b1_ref[...])
    t1 = _dot(h0, wc1_ref[...])
    deg = deg_ref[0, pl.ds(i * RB, RB)] + deg_ref[1, pl.ds(i * RB, RB)] + 1.0
    dinv = lax.rsqrt(deg).reshape(RB, 1)
    u = t1 * dinv
    t1_ref[...] = t1
    ulo_ref[...] = u[:, :HALF]
    uhi_ref[...] = u[:, HALF:]
    dinv_ref[...] = dinv


def _tc_mid_body(t_ref, slo_ref, shi_ref, dinv_ref, bc_ref, g_ref, b_ref,
                 wc_ref, t2_ref, ulo_ref, uhi_ref):
    dinv = dinv_ref[...]
    S = jnp.concatenate([slo_ref[...], shi_ref[...]], axis=1)
    agg = dinv * S + (dinv * dinv) * t_ref[...] + bc_ref[...]
    m = jnp.mean(agg, axis=1, keepdims=True)
    ctr = agg - m
    v = jnp.mean(ctr * ctr, axis=1, keepdims=True)
    h = ctr * lax.rsqrt(v + 1e-5) * g_ref[...] + b_ref[...]
    h = _leaky(h)
    t2 = _dot(h, wc_ref[...])
    u = t2 * dinv
    t2_ref[...] = t2
    ulo_ref[...] = u[:, :HALF]
    uhi_ref[...] = u[:, HALF:]


def _tc_c_body(t_ref, slo_ref, shi_ref, dinv_ref, bc_ref, g_ref, b_ref,
               wr_ref, br_ref, out_ref):
    dinv = dinv_ref[...]
    S = jnp.concatenate([slo_ref[...], shi_ref[...]], axis=1)
    agg = dinv * S + (dinv * dinv) * t_ref[...] + bc_ref[...]
    m = jnp.mean(agg, axis=1, keepdims=True)
    ctr = agg - m
    v = jnp.mean(ctr * ctr, axis=1, keepdims=True)
    h = ctr * lax.rsqrt(v + 1e-5) * g_ref[...] + b_ref[...]
    h = _leaky(h)
    out_ref[...] = _dot(h, wr_ref[...]) + br_ref[...]


def _row_spec(cols):
    return pl.BlockSpec((RB, cols), lambda i: (i, 0))


def _full_spec(shape):
    nd = len(shape)
    return pl.BlockSpec(shape, lambda i: (0,) * nd)


def kernel(x, edge_index, W1, b1, Wc1, bc1, Wc2, bc2, ln_g, ln_b, Wr, br):
    src = edge_index[0]
    dst = edge_index[1]
    pad = jnp.full((EPAD - E,), N, jnp.int32)
    src_full = jnp.concatenate([src, pad])
    dst_full = jnp.concatenate([dst, pad])
    dst_r = dst_full.reshape(NTILES, NCH, CHUNK)
    src_r2 = src_full.reshape(NS, NCH2 // NCH, NCH, CHUNK)
    dst_r2 = dst_full.reshape(NS, NCH2 // NCH, NCH, CHUNK)
    x_pad = jnp.pad(x, ((0, NPAD - N), (0, 0)))
    b1r = b1.reshape(1, H)
    bc1r = bc1.reshape(1, H)
    bc2r = bc2.reshape(1, H)
    gr = ln_g.reshape(1, H)
    br2 = ln_b.reshape(1, H)
    brr = br.reshape(1, OUT)

    sc_degree, sc_segsum = _sc_kernels()
    deg2 = sc_degree(dst_r)

    t1, u1lo, u1hi, dinv = pl.pallas_call(
        _tc_a_body,
        grid=(GRID,),
        in_specs=[_row_spec(D), _full_spec((D, H)), _full_spec((1, H)),
                  _full_spec((H, H)), _full_spec((NC, NPAD))],
        out_specs=[_row_spec(H), _row_spec(HALF), _row_spec(HALF),
                   _row_spec(1)],
        out_shape=[jax.ShapeDtypeStruct((NPAD, H), _F32),
                   jax.ShapeDtypeStruct((NPAD, HALF), _F32),
                   jax.ShapeDtypeStruct((NPAD, HALF), _F32),
                   jax.ShapeDtypeStruct((NPAD, 1), _F32)],
    )(x_pad, W1, b1r, Wc1, deg2)

    s1lo, s1hi = sc_segsum(u1lo, u1hi, src_r2, dst_r2)

    t2, u2lo, u2hi = pl.pallas_call(
        _tc_mid_body,
        grid=(GRID,),
        in_specs=[_row_spec(H), _row_spec(HALF), _row_spec(HALF),
                  _row_spec(1), _full_spec((1, H)), _full_spec((1, H)),
                  _full_spec((1, H)), _full_spec((H, H))],
        out_specs=[_row_spec(H), _row_spec(HALF), _row_spec(HALF)],
        out_shape=[jax.ShapeDtypeStruct((NPAD, H), _F32),
                   jax.ShapeDtypeStruct((NPAD, HALF), _F32),
                   jax.ShapeDtypeStruct((NPAD, HALF), _F32)],
    )(t1, s1lo, s1hi, dinv, bc1r, gr, br2, Wc2)

    s2lo, s2hi = sc_segsum(u2lo, u2hi, src_r2, dst_r2)

    out_pad = pl.pallas_call(
        _tc_c_body,
        grid=(GRID,),
        in_specs=[_row_spec(H), _row_spec(HALF), _row_spec(HALF),
                  _row_spec(1), _full_spec((1, H)), _full_spec((1, H)),
                  _full_spec((1, H)), _full_spec((H, OUT)),
                  _full_spec((1, OUT))],
        out_specs=_row_spec(OUT),
        out_shape=jax.ShapeDtypeStruct((NPAD, OUT), _F32),
    )(t2, s2lo, s2hi, dinv, bc2r, gr, br2, Wr, brr)

    return out_pad[:N]


# speedup vs baseline: 7.6829x; 1.0029x over previous
import functools

import jax
import jax.numpy as jnp
from jax import lax
from jax.experimental import pallas as pl
from jax.experimental.pallas import tpu as pltpu
from jax.experimental.pallas import tpu_sc as plsc

N = 10000
E = 160000
D = 256
H = 256
OUT = 3

NC = 2
NS = 16
NTILES = NC * NS
CHUNK = 128
NCH = 40
EPAD = NTILES * NCH * CHUNK
NCH2 = EPAD // (NS * CHUNK)
NPAD = 10240
HALF = H // 2
RPT = NPAD // NS

def _sc_degree_body(dst_hbm, out_hbm, idx_v, ones_v, zero_v, acc_sh):
    c = lax.axis_index("c")
    s = lax.axis_index("s")
    w = s * NC + c

    for k in range(CHUNK // 16):
        ones_v[pl.ds(k * 16, 16)] = jnp.ones((16,), jnp.float32)

    def _z(i, _):
        zero_v[pl.ds(i * 16, 16)] = jnp.zeros((16,), jnp.float32)
        return 0
    lax.fori_loop(0, RPT // 16, _z, 0)

    pltpu.sync_copy(zero_v, acc_sh.at[pl.ds(s * RPT, RPT)])
    plsc.subcore_barrier()

    pltpu.sync_copy(dst_hbm.at[w], idx_v)

    def _step(j, _):
        pltpu.sync_copy(ones_v, acc_sh.at[idx_v.at[j]], add=True)
        return 0
    lax.fori_loop(0, NCH, _step, 0)

    plsc.subcore_barrier()
    pltpu.sync_copy(acc_sh.at[pl.ds(s * RPT, RPT)],
                    out_hbm.at[c, pl.ds(s * RPT, RPT)])


def _sc_segsum_body(ulo_hbm, uhi_hbm, src_hbm, dst_hbm, outlo_hbm, outhi_hbm,
                    idx_s, idx_d, rows, acc_sh, gsem, ssem):
    c = lax.axis_index("c")
    s = lax.axis_index("s")

    def _z(i, _):
        for k in range(HALF // 16):
            rows[0, i, pl.ds(k * 16, 16)] = jnp.zeros((16,), jnp.float32)
        return 0
    lax.fori_loop(0, CHUNK, _z, 0)
    for t in range(RPT // CHUNK):
        pltpu.sync_copy(rows.at[0], acc_sh.at[pl.ds(s * RPT + t * CHUNK, CHUNK)])
    plsc.subcore_barrier()

    def _run(table, out_hbm):
        for p in range(NCH2 // NCH):
            pltpu.sync_copy(src_hbm.at[s, p], idx_s)
            pltpu.sync_copy(dst_hbm.at[s, p], idx_d)

            pltpu.async_copy(table.at[idx_s.at[0]], rows.at[0], gsem)

            def _pair(i, _):
                j0 = i * 2
                for b in range(2):
                    j = j0 + b

                    @pl.when(j + 1 < NCH)
                    def _():
                        @pl.when(j >= 1)
                        def _():
                            pltpu.make_async_copy(
                                rows.at[1 - b], acc_sh.at[idx_d.at[j - 1]],
                                ssem).wait()
                        pltpu.async_copy(table.at[idx_s.at[j + 1]],
                                         rows.at[1 - b], gsem)

                    pltpu.make_async_copy(table.at[idx_s.at[j]], rows.at[b],
                                          gsem).wait()
                    pltpu.async_copy(rows.at[b], acc_sh.at[idx_d.at[j]],
                                     ssem, add=True)
                return 0
            lax.fori_loop(0, NCH // 2, _pair, 0)

            for b in range(2):
                pltpu.make_async_copy(rows.at[b], acc_sh.at[idx_d.at[0]],
                                      ssem).wait()

        plsc.subcore_barrier()
        for t in range(RPT // CHUNK):
            r0 = s * RPT + t * CHUNK
            pltpu.sync_copy(acc_sh.at[pl.ds(r0, CHUNK)],
                            out_hbm.at[pl.ds(r0, CHUNK)])

    @pl.when(c == 0)
    def _():
        _run(ulo_hbm, outlo_hbm)

    @pl.when(c == 1)
    def _():
        _run(uhi_hbm, outhi_hbm)


@functools.cache
def _sc_kernels():
    mesh = plsc.VectorSubcoreMesh(core_axis_name="c", subcore_axis_name="s")
    sc_degree = functools.partial(
        pl.kernel,
        out_type=jax.ShapeDtypeStruct((NC, NPAD), jnp.float32),
        mesh=mesh,
        scratch_types=[
            pltpu.VMEM((NCH, CHUNK), jnp.int32),
            pltpu.VMEM((CHUNK,), jnp.float32),
            pltpu.VMEM((RPT,), jnp.float32),
            pltpu.VMEM_SHARED((NPAD,), jnp.float32),
        ],
    )(_sc_degree_body)
    sc_segsum = functools.partial(
        pl.kernel,
        out_type=(jax.ShapeDtypeStruct((NPAD, HALF), jnp.float32),
                  jax.ShapeDtypeStruct((NPAD, HALF), jnp.float32)),
        mesh=mesh,
        scratch_types=[
            pltpu.VMEM((NCH, CHUNK), jnp.int32),
            pltpu.VMEM((NCH, CHUNK), jnp.int32),
            pltpu.VMEM((2, CHUNK, HALF), jnp.float32),
            pltpu.VMEM_SHARED((NPAD, HALF), jnp.float32),
            pltpu.SemaphoreType.DMA,
            pltpu.SemaphoreType.DMA,
        ],
    )(_sc_segsum_body)
    return sc_degree, sc_segsum


RB = 1024
GRID = NPAD // RB
_F32 = jnp.float32


def _leaky(x):
    return jnp.where(x >= 0, x, 0.01 * x)


def _dot(a, b):
    return lax.dot_general(a, b, (((1,), (0,)), ((), ())),
                           precision=lax.Precision.HIGHEST,
                           preferred_element_type=_F32)


def _tc_a1_body(x_ref, w1_ref, b1_ref, wc1_ref, t1_ref):
    h0 = _leaky(_dot(x_ref[...], w1_ref[...]) + b1_ref[...])
    t1_ref[...] = _dot(h0, wc1_ref[...])


def _tc_a2_body(t1_ref, deg_ref, ulo_ref, uhi_ref, dinv_ref):
    i = pl.program_id(0)
    deg = deg_ref[0, pl.ds(i * RB, RB)] + deg_ref[1, pl.ds(i * RB, RB)] + 1.0
    dinv = lax.rsqrt(deg).reshape(RB, 1)
    u = t1_ref[...] * dinv
    ulo_ref[...] = u[:, :HALF]
    uhi_ref[...] = u[:, HALF:]
    dinv_ref[...] = dinv


def _tc_mid_body(t_ref, slo_ref, shi_ref, dinv_ref, bc_ref, g_ref, b_ref,
                 wc_ref, t2_ref, ulo_ref, uhi_ref):
    dinv = dinv_ref[...]
    S = jnp.concatenate([slo_ref[...], shi_ref[...]], axis=1)
    agg = dinv * S + (dinv * dinv) * t_ref[...] + bc_ref[...]
    m = jnp.mean(agg, axis=1, keepdims=True)
    ctr = agg - m
    v = jnp.mean(ctr * ctr, axis=1, keepdims=True)
    h = ctr * lax.rsqrt(v + 1e-5) * g_ref[...] + b_ref[...]
    h = _leaky(h)
    t2 = _dot(h, wc_ref[...])
    u = t2 * dinv
    t2_ref[...] = t2
    ulo_ref[...] = u[:, :HALF]
    uhi_ref[...] = u[:, HALF:]


def _tc_c_body(t_ref, slo_ref, shi_ref, dinv_ref, bc_ref, g_ref, b_ref,
               wr_ref, br_ref, out_ref):
    dinv = dinv_ref[...]
    S = jnp.concatenate([slo_ref[...], shi_ref[...]], axis=1)
    agg = dinv * S + (dinv * dinv) * t_ref[...] + bc_ref[...]
    m = jnp.mean(agg, axis=1, keepdims=True)
    ctr = agg - m
    v = jnp.mean(ctr * ctr, axis=1, keepdims=True)
    h = ctr * lax.rsqrt(v + 1e-5) * g_ref[...] + b_ref[...]
    h = _leaky(h)
    out_ref[...] = _dot(h, wr_ref[...]) + br_ref[...]


def _row_spec(cols):
    return pl.BlockSpec((RB, cols), lambda i: (i, 0))


def _full_spec(shape):
    nd = len(shape)
    return pl.BlockSpec(shape, lambda i: (0,) * nd)


def kernel(x, edge_index, W1, b1, Wc1, bc1, Wc2, bc2, ln_g, ln_b, Wr, br):
    src = edge_index[0]
    dst = edge_index[1]
    pad = jnp.full((EPAD - E,), N, jnp.int32)
    src_full = jnp.concatenate([src, pad])
    dst_full = jnp.concatenate([dst, pad])
    dst_r = dst_full.reshape(NTILES, NCH, CHUNK)
    src_r2 = src_full.reshape(NS, NCH2 // NCH, NCH, CHUNK)
    dst_r2 = dst_full.reshape(NS, NCH2 // NCH, NCH, CHUNK)
    x_pad = jnp.pad(x, ((0, NPAD - N), (0, 0)))
    b1r = b1.reshape(1, H)
    bc1r = bc1.reshape(1, H)
    bc2r = bc2.reshape(1, H)
    gr = ln_g.reshape(1, H)
    br2 = ln_b.reshape(1, H)
    brr = br.reshape(1, OUT)

    sc_degree, sc_segsum = _sc_kernels()
    deg2 = sc_degree(dst_r)

    t1 = pl.pallas_call(
        _tc_a1_body,
        grid=(GRID,),
        in_specs=[_row_spec(D), _full_spec((D, H)), _full_spec((1, H)),
                  _full_spec((H, H))],
        out_specs=_row_spec(H),
        out_shape=jax.ShapeDtypeStruct((NPAD, H), _F32),
    )(x_pad, W1, b1r, Wc1)

    u1lo, u1hi, dinv = pl.pallas_call(
        _tc_a2_body,
        grid=(GRID,),
        in_specs=[_row_spec(H), _full_spec((NC, NPAD))],
        out_specs=[_row_spec(HALF), _row_spec(HALF), _row_spec(1)],
        out_shape=[jax.ShapeDtypeStruct((NPAD, HALF), _F32),
                   jax.ShapeDtypeStruct((NPAD, HALF), _F32),
                   jax.ShapeDtypeStruct((NPAD, 1), _F32)],
    )(t1, deg2)

    s1lo, s1hi = sc_segsum(u1lo, u1hi, src_r2, dst_r2)

    t2, u2lo, u2hi = pl.pallas_call(
        _tc_mid_body,
        grid=(GRID,),
        in_specs=[_row_spec(H), _row_spec(HALF), _row_spec(HALF),
                  _row_spec(1), _full_spec((1, H)), _full_spec((1, H)),
                  _full_spec((1, H)), _full_spec((H, H))],
        out_specs=[_row_spec(H), _row_spec(HALF), _row_spec(HALF)],
        out_shape=[jax.ShapeDtypeStruct((NPAD, H), _F32),
                   jax.ShapeDtypeStruct((NPAD, HALF), _F32),
                   jax.ShapeDtypeStruct((NPAD, HALF), _F32)],
    )(t1, s1lo, s1hi, dinv, bc1r, gr, br2, Wc2)

    s2lo, s2hi = sc_segsum(u2lo, u2hi, src_r2, dst_r2)

    out_pad = pl.pallas_call(
        _tc_c_body,
        grid=(GRID,),
        in_specs=[_row_spec(H), _row_spec(HALF), _row_spec(HALF),
                  _row_spec(1), _full_spec((1, H)), _full_spec((1, H)),
                  _full_spec((1, H)), _full_spec((H, OUT)),
                  _full_spec((1, OUT))],
        out_specs=_row_spec(OUT),
        out_shape=jax.ShapeDtypeStruct((NPAD, OUT), _F32),
    )(t2, s2lo, s2hi, dinv, bc2r, gr, br2, Wr, brr)

    return out_pad[:N]


# speedup vs baseline: 7.7947x; 1.0145x over previous
import functools

import jax
import jax.numpy as jnp
from jax import lax
from jax.experimental import pallas as pl
from jax.experimental.pallas import tpu as pltpu
from jax.experimental.pallas import tpu_sc as plsc

N = 10000
E = 160000
D = 256
H = 256
OUT = 3

NC = 2
NS = 16
NTILES = NC * NS
CHUNK = 128
NCH = 40
EPAD = NTILES * NCH * CHUNK
NCH2 = EPAD // (NS * CHUNK)
NPAD = 10240
HALF = H // 2
CH2 = CHUNK // 2
PCH = 20
RPT = NPAD // NS

def _sc_degree_body(dst_hbm, out_hbm, idx_v, ones_v, zero_v, acc_sh):
    c = lax.axis_index("c")
    s = lax.axis_index("s")
    w = s * NC + c

    for k in range(CHUNK // 16):
        ones_v[pl.ds(k * 16, 16)] = jnp.ones((16,), jnp.float32)

    def _z(i, _):
        zero_v[pl.ds(i * 16, 16)] = jnp.zeros((16,), jnp.float32)
        return 0
    lax.fori_loop(0, RPT // 16, _z, 0)

    pltpu.sync_copy(zero_v, acc_sh.at[pl.ds(s * RPT, RPT)])
    plsc.subcore_barrier()

    pltpu.sync_copy(dst_hbm.at[w], idx_v)

    def _step(j, _):
        pltpu.sync_copy(ones_v, acc_sh.at[idx_v.at[j]], add=True)
        return 0
    lax.fori_loop(0, NCH, _step, 0)

    plsc.subcore_barrier()
    pltpu.sync_copy(acc_sh.at[pl.ds(s * RPT, RPT)],
                    out_hbm.at[c, pl.ds(s * RPT, RPT)])


def _sc_segsum_body(ulo_hbm, uhi_hbm, src_hbm, dst_hbm, outlo_hbm, outhi_hbm,
                    idx_s, idx_d, rows, acc_sh, gsem, ssem):
    c = lax.axis_index("c")
    s = lax.axis_index("s")

    def _z(i, _):
        for k in range(HALF // 16):
            rows[0, i, pl.ds(k * 16, 16)] = jnp.zeros((16,), jnp.float32)
        return 0
    lax.fori_loop(0, CH2, _z, 0)
    for t in range(RPT // CH2):
        pltpu.sync_copy(rows.at[0], acc_sh.at[pl.ds(s * RPT + t * CH2, CH2)])
    plsc.subcore_barrier()

    def _run(table, out_hbm):
        def _g2(j, b):
            pltpu.async_copy(table.at[idx_s.at[2 * j]], rows.at[2 * b], gsem)
            pltpu.async_copy(table.at[idx_s.at[2 * j + 1]], rows.at[2 * b + 1],
                             gsem)

        def _wg2(j, b):
            pltpu.make_async_copy(table.at[idx_s.at[2 * j]], rows.at[2 * b],
                                  gsem).wait()
            pltpu.make_async_copy(table.at[idx_s.at[2 * j + 1]],
                                  rows.at[2 * b + 1], gsem).wait()

        def _s2(j, b):
            pltpu.async_copy(rows.at[2 * b], acc_sh.at[idx_d.at[2 * j]],
                             ssem, add=True)
            pltpu.async_copy(rows.at[2 * b + 1], acc_sh.at[idx_d.at[2 * j + 1]],
                             ssem, add=True)

        def _ws2(j, b):
            pltpu.make_async_copy(rows.at[2 * b], acc_sh.at[idx_d.at[2 * j]],
                                  ssem).wait()
            pltpu.make_async_copy(rows.at[2 * b + 1],
                                  acc_sh.at[idx_d.at[2 * j + 1]], ssem).wait()

        for p in range(NCH2 // PCH):
            pltpu.sync_copy(src_hbm.at[s, p], idx_s)
            pltpu.sync_copy(dst_hbm.at[s, p], idx_d)

            _g2(0, 0)

            def _pair(i, _):
                j0 = i * 2
                for b in range(2):
                    j = j0 + b

                    @pl.when(j + 1 < PCH)
                    def _():
                        @pl.when(j >= 1)
                        def _():
                            _ws2(j - 1, 1 - b)
                        _g2(j + 1, 1 - b)

                    _wg2(j, b)
                    _s2(j, b)
                return 0
            lax.fori_loop(0, PCH // 2, _pair, 0)

            _ws2(PCH - 2, 0)
            _ws2(PCH - 1, 1)

        plsc.subcore_barrier()
        for t in range(RPT // CHUNK):
            r0 = s * RPT + t * CHUNK
            pltpu.sync_copy(acc_sh.at[pl.ds(r0, CHUNK)],
                            out_hbm.at[pl.ds(r0, CHUNK)])

    @pl.when(c == 0)
    def _():
        _run(ulo_hbm, outlo_hbm)

    @pl.when(c == 1)
    def _():
        _run(uhi_hbm, outhi_hbm)


@functools.cache
def _sc_kernels():
    mesh = plsc.VectorSubcoreMesh(core_axis_name="c", subcore_axis_name="s")
    sc_degree = functools.partial(
        pl.kernel,
        out_type=jax.ShapeDtypeStruct((NC, NPAD), jnp.float32),
        mesh=mesh,
        scratch_types=[
            pltpu.VMEM((NCH, CHUNK), jnp.int32),
            pltpu.VMEM((CHUNK,), jnp.float32),
            pltpu.VMEM((RPT,), jnp.float32),
            pltpu.VMEM_SHARED((NPAD,), jnp.float32),
        ],
    )(_sc_degree_body)
    sc_segsum = functools.partial(
        pl.kernel,
        out_type=(jax.ShapeDtypeStruct((NPAD, HALF), jnp.float32),
                  jax.ShapeDtypeStruct((NPAD, HALF), jnp.float32)),
        mesh=mesh,
        scratch_types=[
            pltpu.VMEM((2 * PCH, CH2), jnp.int32),
            pltpu.VMEM((2 * PCH, CH2), jnp.int32),
            pltpu.VMEM((4, CHUNK // 2, HALF), jnp.float32),
            pltpu.VMEM_SHARED((NPAD, HALF), jnp.float32),
            pltpu.SemaphoreType.DMA,
            pltpu.SemaphoreType.DMA,
        ],
    )(_sc_segsum_body)
    return sc_degree, sc_segsum


RB = 1024
GRID = NPAD // RB
_F32 = jnp.float32


def _leaky(x):
    return jnp.where(x >= 0, x, 0.01 * x)


def _dot(a, b):
    return lax.dot_general(a, b, (((1,), (0,)), ((), ())),
                           precision=lax.Precision.HIGHEST,
                           preferred_element_type=_F32)


def _tc_a1_body(x_ref, w1_ref, b1_ref, wc1_ref, t1_ref):
    h0 = _leaky(_dot(x_ref[...], w1_ref[...]) + b1_ref[...])
    t1_ref[...] = _dot(h0, wc1_ref[...])


def _tc_a2_body(t1_ref, deg_ref, ulo_ref, uhi_ref, dinv_ref):
    i = pl.program_id(0)
    deg = deg_ref[0, pl.ds(i * RB, RB)] + deg_ref[1, pl.ds(i * RB, RB)] + 1.0
    dinv = lax.rsqrt(deg).reshape(RB, 1)
    u = t1_ref[...] * dinv
    ulo_ref[...] = u[:, :HALF]
    uhi_ref[...] = u[:, HALF:]
    dinv_ref[...] = dinv


def _tc_mid_body(t_ref, slo_ref, shi_ref, dinv_ref, bc_ref, g_ref, b_ref,
                 wc_ref, t2_ref, ulo_ref, uhi_ref):
    dinv = dinv_ref[...]
    S = jnp.concatenate([slo_ref[...], shi_ref[...]], axis=1)
    agg = dinv * S + (dinv * dinv) * t_ref[...] + bc_ref[...]
    m = jnp.mean(agg, axis=1, keepdims=True)
    ctr = agg - m
    v = jnp.mean(ctr * ctr, axis=1, keepdims=True)
    h = ctr * lax.rsqrt(v + 1e-5) * g_ref[...] + b_ref[...]
    h = _leaky(h)
    t2 = _dot(h, wc_ref[...])
    u = t2 * dinv
    t2_ref[...] = t2
    ulo_ref[...] = u[:, :HALF]
    uhi_ref[...] = u[:, HALF:]


def _tc_c_body(t_ref, slo_ref, shi_ref, dinv_ref, bc_ref, g_ref, b_ref,
               wr_ref, br_ref, out_ref):
    dinv = dinv_ref[...]
    S = jnp.concatenate([slo_ref[...], shi_ref[...]], axis=1)
    agg = dinv * S + (dinv * dinv) * t_ref[...] + bc_ref[...]
    m = jnp.mean(agg, axis=1, keepdims=True)
    ctr = agg - m
    v = jnp.mean(ctr * ctr, axis=1, keepdims=True)
    h = ctr * lax.rsqrt(v + 1e-5) * g_ref[...] + b_ref[...]
    h = _leaky(h)
    out_ref[...] = _dot(h, wr_ref[...]) + br_ref[...]


def _row_spec(cols):
    return pl.BlockSpec((RB, cols), lambda i: (i, 0))


def _full_spec(shape):
    nd = len(shape)
    return pl.BlockSpec(shape, lambda i: (0,) * nd)


def kernel(x, edge_index, W1, b1, Wc1, bc1, Wc2, bc2, ln_g, ln_b, Wr, br):
    src = edge_index[0]
    dst = edge_index[1]
    pad = jnp.full((EPAD - E,), N, jnp.int32)
    src_full = jnp.concatenate([src, pad])
    dst_full = jnp.concatenate([dst, pad])
    dst_r = dst_full.reshape(NTILES, NCH, CHUNK)
    src_r2 = src_full.reshape(NS, NCH2 // PCH, 2 * PCH, CH2)
    dst_r2 = dst_full.reshape(NS, NCH2 // PCH, 2 * PCH, CH2)
    x_pad = jnp.pad(x, ((0, NPAD - N), (0, 0)))
    b1r = b1.reshape(1, H)
    bc1r = bc1.reshape(1, H)
    bc2r = bc2.reshape(1, H)
    gr = ln_g.reshape(1, H)
    br2 = ln_b.reshape(1, H)
    brr = br.reshape(1, OUT)

    sc_degree, sc_segsum = _sc_kernels()
    deg2 = sc_degree(dst_r)

    t1 = pl.pallas_call(
        _tc_a1_body,
        grid=(GRID,),
        in_specs=[_row_spec(D), _full_spec((D, H)), _full_spec((1, H)),
                  _full_spec((H, H))],
        out_specs=_row_spec(H),
        out_shape=jax.ShapeDtypeStruct((NPAD, H), _F32),
    )(x_pad, W1, b1r, Wc1)

    u1lo, u1hi, dinv = pl.pallas_call(
        _tc_a2_body,
        grid=(GRID,),
        in_specs=[_row_spec(H), _full_spec((NC, NPAD))],
        out_specs=[_row_spec(HALF), _row_spec(HALF), _row_spec(1)],
        out_shape=[jax.ShapeDtypeStruct((NPAD, HALF), _F32),
                   jax.ShapeDtypeStruct((NPAD, HALF), _F32),
                   jax.ShapeDtypeStruct((NPAD, 1), _F32)],
    )(t1, deg2)

    s1lo, s1hi = sc_segsum(u1lo, u1hi, src_r2, dst_r2)

    t2, u2lo, u2hi = pl.pallas_call(
        _tc_mid_body,
        grid=(GRID,),
        in_specs=[_row_spec(H), _row_spec(HALF), _row_spec(HALF),
                  _row_spec(1), _full_spec((1, H)), _full_spec((1, H)),
                  _full_spec((1, H)), _full_spec((H, H))],
        out_specs=[_row_spec(H), _row_spec(HALF), _row_spec(HALF)],
        out_shape=[jax.ShapeDtypeStruct((NPAD, H), _F32),
                   jax.ShapeDtypeStruct((NPAD, HALF), _F32),
                   jax.ShapeDtypeStruct((NPAD, HALF), _F32)],
    )(t1, s1lo, s1hi, dinv, bc1r, gr, br2, Wc2)

    s2lo, s2hi = sc_segsum(u2lo, u2hi, src_r2, dst_r2)

    out_pad = pl.pallas_call(
        _tc_c_body,
        grid=(GRID,),
        in_specs=[_row_spec(H), _row_spec(HALF), _row_spec(HALF),
                  _row_spec(1), _full_spec((1, H)), _full_spec((1, H)),
                  _full_spec((1, H)), _full_spec((H, OUT)),
                  _full_spec((1, OUT))],
        out_specs=_row_spec(OUT),
        out_shape=jax.ShapeDtypeStruct((NPAD, OUT), _F32),
    )(t2, s2lo, s2hi, dinv, bc2r, gr, br2, Wr, brr)

    return out_pad[:N]


# speedup vs baseline: 8.1460x; 1.0451x over previous
import functools

import jax
import jax.numpy as jnp
from jax import lax
from jax.experimental import pallas as pl
from jax.experimental.pallas import tpu as pltpu
from jax.experimental.pallas import tpu_sc as plsc

N = 10000
E = 160000
D = 256
H = 256
OUT = 3

NC = 2
NS = 16
NTILES = NC * NS
CHUNK = 128
NCH = 40
EPAD = NTILES * NCH * CHUNK
NCH2 = EPAD // (NS * CHUNK)
NPAD = 10240
HALF = H // 2
CH2 = CHUNK // 2
PCH = 20
RPT = NPAD // NS

def _sc_degree_body(dst_hbm, out_hbm, idx_v, ones_v, zero_v, acc_sh):
    c = lax.axis_index("c")
    s = lax.axis_index("s")
    w = s * NC + c

    for k in range(CHUNK // 16):
        ones_v[pl.ds(k * 16, 16)] = jnp.ones((16,), jnp.float32)

    def _z(i, _):
        zero_v[pl.ds(i * 16, 16)] = jnp.zeros((16,), jnp.float32)
        return 0
    lax.fori_loop(0, RPT // 16, _z, 0)

    pltpu.sync_copy(zero_v, acc_sh.at[pl.ds(s * RPT, RPT)])
    plsc.subcore_barrier()

    pltpu.sync_copy(dst_hbm.at[w], idx_v)

    def _step(j, _):
        pltpu.sync_copy(ones_v, acc_sh.at[idx_v.at[j]], add=True)
        return 0
    lax.fori_loop(0, NCH, _step, 0)

    plsc.subcore_barrier()
    pltpu.sync_copy(acc_sh.at[pl.ds(s * RPT, RPT)],
                    out_hbm.at[c, pl.ds(s * RPT, RPT)])


def _sc_segsum_body(ulo_hbm, uhi_hbm, src_hbm, dst_hbm, outlo_hbm, outhi_hbm,
                    idx_s, idx_d, rows, acc_sh, gsem, ssem):
    c = lax.axis_index("c")
    s = lax.axis_index("s")

    def _z(i, _):
        for k in range(HALF // 16):
            rows[0, i, pl.ds(k * 16, 16)] = jnp.zeros((16,), jnp.float32)
        return 0
    lax.fori_loop(0, CH2, _z, 0)
    for t in range(RPT // CH2):
        pltpu.sync_copy(rows.at[0], acc_sh.at[pl.ds(s * RPT + t * CH2, CH2)])
    plsc.subcore_barrier()

    def _run(table, out_hbm):
        def _g2(j, b):
            pltpu.async_copy(table.at[idx_s.at[2 * j]], rows.at[2 * b], gsem)
            pltpu.async_copy(table.at[idx_s.at[2 * j + 1]], rows.at[2 * b + 1],
                             gsem)

        def _wg2(j, b):
            pltpu.make_async_copy(table.at[idx_s.at[2 * j]], rows.at[2 * b],
                                  gsem).wait()
            pltpu.make_async_copy(table.at[idx_s.at[2 * j + 1]],
                                  rows.at[2 * b + 1], gsem).wait()

        def _s2(j, b):
            pltpu.async_copy(rows.at[2 * b], acc_sh.at[idx_d.at[2 * j]],
                             ssem, add=True)
            pltpu.async_copy(rows.at[2 * b + 1], acc_sh.at[idx_d.at[2 * j + 1]],
                             ssem, add=True)

        def _ws2(j, b):
            pltpu.make_async_copy(rows.at[2 * b], acc_sh.at[idx_d.at[2 * j]],
                                  ssem).wait()
            pltpu.make_async_copy(rows.at[2 * b + 1],
                                  acc_sh.at[idx_d.at[2 * j + 1]], ssem).wait()

        for p in range(NCH2 // PCH):
            pltpu.sync_copy(src_hbm.at[s, p], idx_s)
            pltpu.sync_copy(dst_hbm.at[s, p], idx_d)

            _g2(0, 0)

            def _pair(i, _):
                j0 = i * 2
                for b in range(2):
                    j = j0 + b

                    @pl.when(j + 1 < PCH)
                    def _():
                        @pl.when(j >= 1)
                        def _():
                            _ws2(j - 1, 1 - b)
                        _g2(j + 1, 1 - b)

                    _wg2(j, b)
                    _s2(j, b)
                return 0
            lax.fori_loop(0, PCH // 2, _pair, 0)

            _ws2(PCH - 2, 0)
            _ws2(PCH - 1, 1)

        plsc.subcore_barrier()
        for t in range(RPT // CHUNK):
            r0 = s * RPT + t * CHUNK
            pltpu.sync_copy(acc_sh.at[pl.ds(r0, CHUNK)],
                            out_hbm.at[pl.ds(r0, CHUNK)])

    @pl.when(c == 0)
    def _():
        _run(ulo_hbm, outlo_hbm)

    @pl.when(c == 1)
    def _():
        _run(uhi_hbm, outhi_hbm)


@functools.cache
def _sc_kernels():
    mesh = plsc.VectorSubcoreMesh(core_axis_name="c", subcore_axis_name="s")
    sc_degree = functools.partial(
        pl.kernel,
        out_type=jax.ShapeDtypeStruct((NC, NPAD), jnp.float32),
        mesh=mesh,
        scratch_types=[
            pltpu.VMEM((NCH, CHUNK), jnp.int32),
            pltpu.VMEM((CHUNK,), jnp.float32),
            pltpu.VMEM((RPT,), jnp.float32),
            pltpu.VMEM_SHARED((NPAD,), jnp.float32),
        ],
    )(_sc_degree_body)
    sc_segsum = functools.partial(
        pl.kernel,
        out_type=(jax.ShapeDtypeStruct((NPAD, HALF), jnp.float32),
                  jax.ShapeDtypeStruct((NPAD, HALF), jnp.float32)),
        mesh=mesh,
        scratch_types=[
            pltpu.VMEM((2 * PCH, CH2), jnp.int32),
            pltpu.VMEM((2 * PCH, CH2), jnp.int32),
            pltpu.VMEM((4, CHUNK // 2, HALF), jnp.float32),
            pltpu.VMEM_SHARED((NPAD, HALF), jnp.float32),
            pltpu.SemaphoreType.DMA,
            pltpu.SemaphoreType.DMA,
        ],
    )(_sc_segsum_body)
    return sc_degree, sc_segsum


RB = 1024
GRID = NPAD // RB
_F32 = jnp.float32


def _leaky(x):
    return jnp.where(x >= 0, x, 0.01 * x)


def _dot(a, b):
    return lax.dot_general(a, b, (((1,), (0,)), ((), ())),
                           preferred_element_type=_F32)


def _tc_a1_body(x_ref, w1_ref, b1_ref, wc1_ref, t1_ref):
    h0 = _leaky(_dot(x_ref[...], w1_ref[...]) + b1_ref[...])
    t1_ref[...] = _dot(h0, wc1_ref[...])


def _tc_a2_body(t1_ref, deg_ref, ulo_ref, uhi_ref, dinv_ref):
    i = pl.program_id(0)
    deg = deg_ref[0, pl.ds(i * RB, RB)] + deg_ref[1, pl.ds(i * RB, RB)] + 1.0
    dinv = lax.rsqrt(deg).reshape(RB, 1)
    u = t1_ref[...] * dinv
    ulo_ref[...] = u[:, :HALF]
    uhi_ref[...] = u[:, HALF:]
    dinv_ref[...] = dinv


def _tc_mid_body(t_ref, slo_ref, shi_ref, dinv_ref, bc_ref, g_ref, b_ref,
                 wc_ref, t2_ref, ulo_ref, uhi_ref):
    dinv = dinv_ref[...]
    S = jnp.concatenate([slo_ref[...], shi_ref[...]], axis=1)
    agg = dinv * S + (dinv * dinv) * t_ref[...] + bc_ref[...]
    m = jnp.mean(agg, axis=1, keepdims=True)
    ctr = agg - m
    v = jnp.mean(ctr * ctr, axis=1, keepdims=True)
    h = ctr * lax.rsqrt(v + 1e-5) * g_ref[...] + b_ref[...]
    h = _leaky(h)
    t2 = _dot(h, wc_ref[...])
    u = t2 * dinv
    t2_ref[...] = t2
    ulo_ref[...] = u[:, :HALF]
    uhi_ref[...] = u[:, HALF:]


def _tc_c_body(t_ref, slo_ref, shi_ref, dinv_ref, bc_ref, g_ref, b_ref,
               wr_ref, br_ref, out_ref):
    dinv = dinv_ref[...]
    S = jnp.concatenate([slo_ref[...], shi_ref[...]], axis=1)
    agg = dinv * S + (dinv * dinv) * t_ref[...] + bc_ref[...]
    m = jnp.mean(agg, axis=1, keepdims=True)
    ctr = agg - m
    v = jnp.mean(ctr * ctr, axis=1, keepdims=True)
    h = ctr * lax.rsqrt(v + 1e-5) * g_ref[...] + b_ref[...]
    h = _leaky(h)
    out_ref[...] = _dot(h, wr_ref[...]) + br_ref[...]


def _row_spec(cols):
    return pl.BlockSpec((RB, cols), lambda i: (i, 0))


def _full_spec(shape):
    nd = len(shape)
    return pl.BlockSpec(shape, lambda i: (0,) * nd)


def kernel(x, edge_index, W1, b1, Wc1, bc1, Wc2, bc2, ln_g, ln_b, Wr, br):
    src = edge_index[0]
    dst = edge_index[1]
    pad = jnp.full((EPAD - E,), N, jnp.int32)
    src_full = jnp.concatenate([src, pad])
    dst_full = jnp.concatenate([dst, pad])
    dst_r = dst_full.reshape(NTILES, NCH, CHUNK)
    src_r2 = src_full.reshape(NS, NCH2 // PCH, 2 * PCH, CH2)
    dst_r2 = dst_full.reshape(NS, NCH2 // PCH, 2 * PCH, CH2)
    x_pad = jnp.pad(x, ((0, NPAD - N), (0, 0)))
    b1r = b1.reshape(1, H)
    bc1r = bc1.reshape(1, H)
    bc2r = bc2.reshape(1, H)
    gr = ln_g.reshape(1, H)
    br2 = ln_b.reshape(1, H)
    brr = br.reshape(1, OUT)

    sc_degree, sc_segsum = _sc_kernels()
    deg2 = sc_degree(dst_r)

    t1 = pl.pallas_call(
        _tc_a1_body,
        grid=(GRID,),
        in_specs=[_row_spec(D), _full_spec((D, H)), _full_spec((1, H)),
                  _full_spec((H, H))],
        out_specs=_row_spec(H),
        out_shape=jax.ShapeDtypeStruct((NPAD, H), _F32),
    )(x_pad, W1, b1r, Wc1)

    u1lo, u1hi, dinv = pl.pallas_call(
        _tc_a2_body,
        grid=(GRID,),
        in_specs=[_row_spec(H), _full_spec((NC, NPAD))],
        out_specs=[_row_spec(HALF), _row_spec(HALF), _row_spec(1)],
        out_shape=[jax.ShapeDtypeStruct((NPAD, HALF), _F32),
                   jax.ShapeDtypeStruct((NPAD, HALF), _F32),
                   jax.ShapeDtypeStruct((NPAD, 1), _F32)],
    )(t1, deg2)

    s1lo, s1hi = sc_segsum(u1lo, u1hi, src_r2, dst_r2)

    t2, u2lo, u2hi = pl.pallas_call(
        _tc_mid_body,
        grid=(GRID,),
        in_specs=[_row_spec(H), _row_spec(HALF), _row_spec(HALF),
                  _row_spec(1), _full_spec((1, H)), _full_spec((1, H)),
                  _full_spec((1, H)), _full_spec((H, H))],
        out_specs=[_row_spec(H), _row_spec(HALF), _row_spec(HALF)],
        out_shape=[jax.ShapeDtypeStruct((NPAD, H), _F32),
                   jax.ShapeDtypeStruct((NPAD, HALF), _F32),
                   jax.ShapeDtypeStruct((NPAD, HALF), _F32)],
    )(t1, s1lo, s1hi, dinv, bc1r, gr, br2, Wc2)

    s2lo, s2hi = sc_segsum(u2lo, u2hi, src_r2, dst_r2)

    out_pad = pl.pallas_call(
        _tc_c_body,
        grid=(GRID,),
        in_specs=[_row_spec(H), _row_spec(HALF), _row_spec(HALF),
                  _row_spec(1), _full_spec((1, H)), _full_spec((1, H)),
                  _full_spec((1, H)), _full_spec((H, OUT)),
                  _full_spec((1, OUT))],
        out_specs=_row_spec(OUT),
        out_shape=jax.ShapeDtypeStruct((NPAD, OUT), _F32),
    )(t2, s2lo, s2hi, dinv, bc2r, gr, br2, Wr, brr)

    return out_pad[:N]


# speedup vs baseline: 8.3452x; 1.0244x over previous
import functools

import jax
import jax.numpy as jnp
from jax import lax
from jax.experimental import pallas as pl
from jax.experimental.pallas import tpu as pltpu
from jax.experimental.pallas import tpu_sc as plsc

N = 10000
E = 160000
D = 256
H = 256
OUT = 3

NC = 2
NS = 16
NTILES = NC * NS
CHUNK = 128
NCH = 40
EPAD = NTILES * NCH * CHUNK
NCH2 = EPAD // (NS * CHUNK)
NPAD = 10240
HALF = H // 2
CH2 = CHUNK // 2
PCH = 20
RPT = NPAD // NS

def _sc_degree_body(dst_hbm, out_hbm, idx_v, ones_v, zero_v, acc_sh):
    c = lax.axis_index("c")
    s = lax.axis_index("s")
    w = s * NC + c

    for k in range(CHUNK // 16):
        ones_v[pl.ds(k * 16, 16)] = jnp.ones((16,), jnp.float32)

    def _z(i, _):
        zero_v[pl.ds(i * 16, 16)] = jnp.zeros((16,), jnp.float32)
        return 0
    lax.fori_loop(0, RPT // 16, _z, 0)

    pltpu.sync_copy(zero_v, acc_sh.at[pl.ds(s * RPT, RPT)])
    plsc.subcore_barrier()

    pltpu.sync_copy(dst_hbm.at[w], idx_v)

    def _step(j, _):
        pltpu.sync_copy(ones_v, acc_sh.at[idx_v.at[j]], add=True)
        return 0
    lax.fori_loop(0, NCH, _step, 0)

    plsc.subcore_barrier()
    pltpu.sync_copy(acc_sh.at[pl.ds(s * RPT, RPT)],
                    out_hbm.at[c, pl.ds(s * RPT, RPT)])


def _sc_segsum_body(ulo_hbm, uhi_hbm, src_hbm, dst_hbm, outlo_hbm, outhi_hbm,
                    idx_s, idx_d, rows, acc_sh, gsem, ssem):
    c = lax.axis_index("c")
    s = lax.axis_index("s")

    def _z(i, _):
        for k in range(HALF // 16):
            rows[0, i, pl.ds(k * 16, 16)] = jnp.zeros((16,), jnp.float32)
        return 0
    lax.fori_loop(0, CH2, _z, 0)
    for t in range(RPT // CH2):
        pltpu.sync_copy(rows.at[0], acc_sh.at[pl.ds(s * RPT + t * CH2, CH2)])
    plsc.subcore_barrier()

    def _run(table, out_hbm):
        def _g2(j, b):
            pltpu.async_copy(table.at[idx_s.at[2 * j]], rows.at[2 * b], gsem)
            pltpu.async_copy(table.at[idx_s.at[2 * j + 1]], rows.at[2 * b + 1],
                             gsem)

        def _wg2(j, b):
            pltpu.make_async_copy(table.at[idx_s.at[2 * j]], rows.at[2 * b],
                                  gsem).wait()
            pltpu.make_async_copy(table.at[idx_s.at[2 * j + 1]],
                                  rows.at[2 * b + 1], gsem).wait()

        def _s2(j, b):
            pltpu.async_copy(rows.at[2 * b], acc_sh.at[idx_d.at[2 * j]],
                             ssem, add=True)
            pltpu.async_copy(rows.at[2 * b + 1], acc_sh.at[idx_d.at[2 * j + 1]],
                             ssem, add=True)

        def _ws2(j, b):
            pltpu.make_async_copy(rows.at[2 * b], acc_sh.at[idx_d.at[2 * j]],
                                  ssem).wait()
            pltpu.make_async_copy(rows.at[2 * b + 1],
                                  acc_sh.at[idx_d.at[2 * j + 1]], ssem).wait()

        for p in range(NCH2 // PCH):
            pltpu.sync_copy(src_hbm.at[s, p], idx_s)
            pltpu.sync_copy(dst_hbm.at[s, p], idx_d)

            _g2(0, 0)

            def _pair(i, _):
                j0 = i * 2
                for b in range(2):
                    j = j0 + b

                    @pl.when(j + 1 < PCH)
                    def _():
                        @pl.when(j >= 1)
                        def _():
                            _ws2(j - 1, 1 - b)
                        _g2(j + 1, 1 - b)

                    _wg2(j, b)
                    _s2(j, b)
                return 0
            lax.fori_loop(0, PCH // 2, _pair, 0)

            _ws2(PCH - 2, 0)
            _ws2(PCH - 1, 1)

        plsc.subcore_barrier()
        for t in range(RPT // CHUNK):
            r0 = s * RPT + t * CHUNK
            pltpu.sync_copy(acc_sh.at[pl.ds(r0, CHUNK)],
                            out_hbm.at[pl.ds(r0, CHUNK)])

    @pl.when(c == 0)
    def _():
        _run(ulo_hbm, outlo_hbm)

    @pl.when(c == 1)
    def _():
        _run(uhi_hbm, outhi_hbm)


@functools.cache
def _sc_kernels():
    mesh = plsc.VectorSubcoreMesh(core_axis_name="c", subcore_axis_name="s")
    sc_degree = functools.partial(
        pl.kernel,
        out_type=jax.ShapeDtypeStruct((NC, NPAD), jnp.float32),
        mesh=mesh,
        scratch_types=[
            pltpu.VMEM((NCH, CHUNK), jnp.int32),
            pltpu.VMEM((CHUNK,), jnp.float32),
            pltpu.VMEM((RPT,), jnp.float32),
            pltpu.VMEM_SHARED((NPAD,), jnp.float32),
        ],
    )(_sc_degree_body)
    sc_segsum = functools.partial(
        pl.kernel,
        out_type=(jax.ShapeDtypeStruct((NPAD, HALF), jnp.float32),
                  jax.ShapeDtypeStruct((NPAD, HALF), jnp.float32)),
        mesh=mesh,
        scratch_types=[
            pltpu.VMEM((2 * PCH, CH2), jnp.int32),
            pltpu.VMEM((2 * PCH, CH2), jnp.int32),
            pltpu.VMEM((4, CHUNK // 2, HALF), jnp.float32),
            pltpu.VMEM_SHARED((NPAD, HALF), jnp.float32),
            pltpu.SemaphoreType.DMA,
            pltpu.SemaphoreType.DMA,
        ],
    )(_sc_segsum_body)
    return sc_degree, sc_segsum


RB = 1024
GRID = NPAD // RB
_F32 = jnp.float32


def _leaky(x):
    return jnp.where(x >= 0, x, 0.01 * x)


def _dot(a, b):
    return lax.dot_general(a, b, (((1,), (0,)), ((), ())),
                           preferred_element_type=_F32)


def _tc_a1_body(x_ref, w1_ref, b1_ref, wc1_ref, t1_ref):
    h0 = _leaky(_dot(x_ref[...], w1_ref[...]) + b1_ref[...])
    t1_ref[...] = _dot(h0, wc1_ref[...])


def _tc_a2_body(t1_ref, deg_ref, ulo_ref, uhi_ref, dinv_ref):
    i = pl.program_id(0)
    deg = deg_ref[0, pl.ds(i * RB, RB)] + deg_ref[1, pl.ds(i * RB, RB)] + 1.0
    dinv = lax.rsqrt(deg).reshape(RB, 1)
    u = t1_ref[...] * dinv
    ulo_ref[...] = u[:, :HALF]
    uhi_ref[...] = u[:, HALF:]
    dinv_ref[...] = dinv


def _tc_mid_body(t_ref, slo_ref, shi_ref, dinv_ref, bc_ref, g_ref, b_ref,
                 wc_ref, t2_ref, ulo_ref, uhi_ref):
    dinv = dinv_ref[...]
    S = jnp.concatenate([slo_ref[...], shi_ref[...]], axis=1)
    agg = dinv * S + (dinv * dinv) * t_ref[...] + bc_ref[...]
    m = jnp.mean(agg, axis=1, keepdims=True)
    ctr = agg - m
    v = jnp.mean(ctr * ctr, axis=1, keepdims=True)
    h = ctr * lax.rsqrt(v + 1e-5) * g_ref[...] + b_ref[...]
    h = _leaky(h)
    t2 = _dot(h, wc_ref[...])
    u = t2 * dinv
    t2_ref[...] = t2
    ulo_ref[...] = u[:, :HALF]
    uhi_ref[...] = u[:, HALF:]


def _tc_c_body(t_ref, slo_ref, shi_ref, dinv_ref, bc_ref, g_ref, b_ref,
               wr_ref, br_ref, out_ref):
    dinv = dinv_ref[...]
    S = jnp.concatenate([slo_ref[...], shi_ref[...]], axis=1)
    agg = dinv * S + (dinv * dinv) * t_ref[...] + bc_ref[...]
    m = jnp.mean(agg, axis=1, keepdims=True)
    ctr = agg - m
    v = jnp.mean(ctr * ctr, axis=1, keepdims=True)
    h = ctr * lax.rsqrt(v + 1e-5) * g_ref[...] + b_ref[...]
    h = _leaky(h)
    out_ref[...] = _dot(h, wr_ref[...]) + br_ref[...]


def _row_spec(cols):
    return pl.BlockSpec((RB, cols), lambda i: (i, 0))


def _full_spec(shape):
    nd = len(shape)
    return pl.BlockSpec(shape, lambda i: (0,) * nd)


def kernel(x, edge_index, W1, b1, Wc1, bc1, Wc2, bc2, ln_g, ln_b, Wr, br):
    src = edge_index[0]
    dst = edge_index[1]
    pad = jnp.full((EPAD - E,), N, jnp.int32)
    src_full = jnp.concatenate([src, pad])
    dst_full = jnp.concatenate([dst, pad])
    dst_r = dst_full.reshape(NTILES, NCH, CHUNK)
    src_r2 = src_full.reshape(NS, NCH2 // PCH, 2 * PCH, CH2)
    dst_r2 = dst_full.reshape(NS, NCH2 // PCH, 2 * PCH, CH2)
    b1r = b1.reshape(1, H)
    bc1r = bc1.reshape(1, H)
    bc2r = bc2.reshape(1, H)
    gr = ln_g.reshape(1, H)
    br2 = ln_b.reshape(1, H)
    brr = br.reshape(1, OUT)

    sc_degree, sc_segsum = _sc_kernels()
    deg2 = sc_degree(dst_r)

    t1 = pl.pallas_call(
        _tc_a1_body,
        grid=(GRID,),
        in_specs=[_row_spec(D), _full_spec((D, H)), _full_spec((1, H)),
                  _full_spec((H, H))],
        out_specs=_row_spec(H),
        out_shape=jax.ShapeDtypeStruct((NPAD, H), _F32),
    )(x, W1, b1r, Wc1)

    u1lo, u1hi, dinv = pl.pallas_call(
        _tc_a2_body,
        grid=(GRID,),
        in_specs=[_row_spec(H), _full_spec((NC, NPAD))],
        out_specs=[_row_spec(HALF), _row_spec(HALF), _row_spec(1)],
        out_shape=[jax.ShapeDtypeStruct((NPAD, HALF), _F32),
                   jax.ShapeDtypeStruct((NPAD, HALF), _F32),
                   jax.ShapeDtypeStruct((NPAD, 1), _F32)],
    )(t1, deg2)

    s1lo, s1hi = sc_segsum(u1lo, u1hi, src_r2, dst_r2)

    t2, u2lo, u2hi = pl.pallas_call(
        _tc_mid_body,
        grid=(GRID,),
        in_specs=[_row_spec(H), _row_spec(HALF), _row_spec(HALF),
                  _row_spec(1), _full_spec((1, H)), _full_spec((1, H)),
                  _full_spec((1, H)), _full_spec((H, H))],
        out_specs=[_row_spec(H), _row_spec(HALF), _row_spec(HALF)],
        out_shape=[jax.ShapeDtypeStruct((NPAD, H), _F32),
                   jax.ShapeDtypeStruct((NPAD, HALF), _F32),
                   jax.ShapeDtypeStruct((NPAD, HALF), _F32)],
    )(t1, s1lo, s1hi, dinv, bc1r, gr, br2, Wc2)

    s2lo, s2hi = sc_segsum(u2lo, u2hi, src_r2, dst_r2)

    out_pad = pl.pallas_call(
        _tc_c_body,
        grid=(GRID,),
        in_specs=[_row_spec(H), _row_spec(HALF), _row_spec(HALF),
                  _row_spec(1), _full_spec((1, H)), _full_spec((1, H)),
                  _full_spec((1, H)), _full_spec((H, OUT)),
                  _full_spec((1, OUT))],
        out_specs=_row_spec(OUT),
        out_shape=jax.ShapeDtypeStruct((NPAD, OUT), _F32),
    )(t2, s2lo, s2hi, dinv, bc2r, gr, br2, Wr, brr)

    return out_pad[:N]
